# baseline (speedup 1.0000x reference)
import functools

import jax
import jax.numpy as jnp
from jax import lax
from jax.experimental import pallas as pl
from jax.experimental.pallas import tpu as pltpu
from jax.experimental.pallas import tpu_sc as plsc

NX = 432
NY = 496
NZ = 1
C = 64
B = 4
N = 80000

P = NZ * NY * NX
BP = B * P
NW = 32
S = BP // NW
WPB = NW // B
W = NX
NCHUNK = S // W
KG = W // 16
CH = 8000
NCOORD = N // CH


def _scatter_kernel(feat_hbm, coords_hbm, out_hbm,
                    coords_v, inv_v, rows_v, tile_v,
                    csem, gsem, wsem):
    wid = lax.axis_index("s") * 2 + lax.axis_index("c")
    base = wid * S
    b = wid // WPB
    p_base = (wid % WPB) * S

    iota16 = lax.iota(jnp.int32, 16)
    zeros16i = jnp.zeros((16,), jnp.int32)
    zeros16f = jnp.zeros((16,), jnp.float32)

    def zero_body(k, _):
        inv_v[pl.ds(k * 16, 16)] = zeros16i
        return 0
    lax.fori_loop(0, S // 16, zero_body, 0)

    def coord_chunk(ci, _):
        pltpu.sync_copy(coords_hbm.at[pl.ds(ci * CH * 4, CH * 4)], coords_v)

        def pillar_body(j, _):
            rows = j * 64 + iota16 * 4
            bb = plsc.load_gather(coords_v, [rows])
            zz = plsc.load_gather(coords_v, [rows + 1])
            yy = plsc.load_gather(coords_v, [rows + 2])
            xx = plsc.load_gather(coords_v, [rows + 3])
            flat = bb * P + zz + yy * NX + xx
            local = flat - base
            m = (local >= 0) & (local < S)
            localc = jnp.minimum(jnp.maximum(local, 0), S - 1)
            ids = ci * CH + j * 16 + iota16 + 1
            plsc.store_scatter(inv_v, [localc], ids, mask=m)
            return 0
        lax.fori_loop(0, CH // 16, pillar_body, 0)
        return 0
    lax.fori_loop(0, NCOORD, coord_chunk, 0)

    def chunk_body(i, _):
        p0 = i * W

        def gstart(k, _):
            inv16 = inv_v[pl.ds(p0 + k * 16, 16)]
            idx16 = jnp.maximum(inv16 - 1, 0)
            pltpu.async_copy(feat_hbm.at[idx16],
                             rows_v.at[pl.ds(k * 16, 16), :], gsem)
            return 0
        lax.fori_loop(0, KG, gstart, 0)

        def gwait(k, _):
            pltpu.make_async_copy(feat_hbm.at[zeros16i],
                                  rows_v.at[pl.ds(k * 16, 16), :],
                                  gsem).wait()
            return 0
        lax.fori_loop(0, KG, gwait, 0)

        def pos_body(k, _):
            inv16 = inv_v[pl.ds(p0 + k * 16, 16)]
            valid = inv16 > 0
            rows = k * 16 + iota16

            def ch_body(c, _):
                vals = plsc.load_gather(rows_v, [rows, zeros16i + c])
                vals = jnp.where(valid, vals, zeros16f)
                tile_v[c, pl.ds(k * 16, 16)] = vals
                return 0
            lax.fori_loop(0, C, ch_body, 0)
            return 0
        lax.fori_loop(0, KG, pos_body, 0)

        p_abs = b * C * P + p_base + p0

        def wstart(c, _):
            pltpu.async_copy(tile_v.at[c, :],
                             out_hbm.at[pl.ds(p_abs + c * P, W)], wsem)
            return 0
        lax.fori_loop(0, C, wstart, 0)

        def wwait(c, _):
            pltpu.make_async_copy(tile_v.at[c, :],
                                  out_hbm.at[pl.ds(p_abs + c * P, W)],
                                  wsem).wait()
            return 0
        lax.fori_loop(0, C, wwait, 0)
        return 0
    lax.fori_loop(0, NCHUNK, chunk_body, 0)


@jax.jit
def kernel(pillar_features, voxel_coords):
    coords = jnp.asarray(voxel_coords, jnp.int32).reshape(-1)
    feat = jnp.asarray(pillar_features, jnp.float32)

    mesh = plsc.VectorSubcoreMesh(core_axis_name="c", subcore_axis_name="s")
    run = functools.partial(
        pl.kernel,
        out_type=jax.ShapeDtypeStruct((B * C * P,), jnp.float32),
        mesh=mesh,
        compiler_params=pltpu.CompilerParams(
            needs_layout_passes=False, use_tc_tiling_on_sc=False),
        scratch_types=[
            pltpu.VMEM((CH * 4,), jnp.int32),
            pltpu.VMEM((S,), jnp.int32),
            pltpu.VMEM((W, C), jnp.float32),
            pltpu.VMEM((C, W), jnp.float32),
            pltpu.SemaphoreType.DMA,
            pltpu.SemaphoreType.DMA,
            pltpu.SemaphoreType.DMA,
        ],
    )(_scatter_kernel)
    out = run(feat, coords)
    return out.reshape(B, C * NZ, NY, NX)


# speedup vs baseline: 8.2669x; 8.2669x over previous
import functools

import jax
import jax.numpy as jnp
from jax import lax
from jax.experimental import pallas as pl
from jax.experimental.pallas import tpu as pltpu
from jax.experimental.pallas import tpu_sc as plsc

NX = 432
NY = 496
NZ = 1
C = 64
B = 4
N = 80000

P = NZ * NY * NX
BP = B * P
NW = 32
S = BP // NW
WPB = NW // B
W = NX
NCHUNK = S // W
KG = W // 16
CH = 8000
NCOORD = N // CH
RMAX = W + 16


def _scatter_kernel(feat_hbm, coords_hbm, out_hbm,
                    coords_v, inv_v, rows_v, tile_v, plist, ilist,
                    gsem, wsem):
    wid = lax.axis_index("s") * 2 + lax.axis_index("c")
    base = wid * S
    b = wid // WPB
    p_base = (wid % WPB) * S
    out_base = b * (C * P) + p_base

    iota16 = lax.iota(jnp.int32, 16)
    zeros16i = jnp.zeros((16,), jnp.int32)
    zeros16f = jnp.zeros((16,), jnp.float32)

    def zero_body(k, _):
        inv_v[pl.ds(k * 16, 16)] = zeros16i
        return 0
    lax.fori_loop(0, S // 16, zero_body, 0)

    def tile_zero(k, _):
        tile_v[pl.ds(k * 16, 16)] = zeros16f
        return 0
    lax.fori_loop(0, (C * W) // 16, tile_zero, 0)

    def coord_chunk(ci, _):
        pltpu.sync_copy(coords_hbm.at[pl.ds(ci * CH * 4, CH * 4)], coords_v)

        def pillar_body(j, _):
            for u in range(4):
                rows = j * 256 + u * 64 + iota16 * 4
                bb = plsc.load_gather(coords_v, [rows])
                zz = plsc.load_gather(coords_v, [rows + 1])
                yy = plsc.load_gather(coords_v, [rows + 2])
                xx = plsc.load_gather(coords_v, [rows + 3])
                flat = bb * P + zz + yy * NX + xx
                local = flat - base
                m = (local >= 0) & (local < S)
                localc = jnp.minimum(jnp.maximum(local, 0), S - 1)
                ids = ci * CH + j * 64 + u * 16 + iota16 + 1
                plsc.store_scatter(inv_v, [localc], ids, mask=m)
            return 0
        lax.fori_loop(0, CH // 64, pillar_body, 0)
        return 0
    lax.fori_loop(0, NCOORD, coord_chunk, 0)

    def chunk_body(i, _):
        p0 = i * W

        def cbody(k, cnt):
            inv16 = inv_v[pl.ds(p0 + k * 16, 16)]
            m = inv16 > 0
            plsc.store_compressed(plist.at[pl.ds(cnt, 16)],
                                  k * 16 + iota16, mask=m)
            plsc.store_compressed(ilist.at[pl.ds(cnt, 16)],
                                  inv16 - 1, mask=m)
            return cnt + jnp.sum(m.astype(jnp.int32))
        cnt = lax.fori_loop(0, KG, cbody, 0)
        ilist[pl.ds(cnt, 16)] = zeros16i
        nb = (cnt + 15) // 16

        def gfire(q, _):
            idxv = ilist[pl.ds(q * 16, 16)]
            pltpu.async_copy(feat_hbm.at[idxv],
                             rows_v.at[pl.ds(q * 16, 16), :], gsem)
            return 0
        lax.fori_loop(0, nb, gfire, 0)

        def gdrain(q, _):
            pltpu.make_async_copy(feat_hbm.at[zeros16i],
                                  rows_v.at[pl.ds(q * 16, 16), :],
                                  gsem).wait()
            return 0
        lax.fori_loop(0, nb, gdrain, 0)

        def sbody(q, _):
            pos16 = plist[pl.ds(q * 16, 16)]
            for lane in range(16):
                ok16 = (zeros16i + q * 16 + lane) < cnt
                pj = pos16[lane]
                row = q * 16 + lane
                for g in range(4):
                    vals = rows_v[row, pl.ds(g * 16, 16)]
                    tidx = (g * 16 + iota16) * W + pj
                    plsc.store_scatter(tile_v, [tidx], vals, mask=ok16)
            return 0
        lax.fori_loop(0, nb, sbody, 0)

        off0 = out_base + p0
        for c in range(C):
            pltpu.async_copy(tile_v.at[pl.ds(c * W, W)],
                             out_hbm.at[pl.ds(off0 + c * P, W)], wsem)

        pltpu.make_async_copy(out_hbm.at[pl.ds(0, C * W)], tile_v,
                              wsem).wait()

        def zbody(q, _):
            pos16 = plist[pl.ds(q * 16, 16)]
            for lane in range(16):
                ok16 = (zeros16i + q * 16 + lane) < cnt
                pj = pos16[lane]
                for g in range(4):
                    tidx = (g * 16 + iota16) * W + pj
                    plsc.store_scatter(tile_v, [tidx], zeros16f, mask=ok16)
            return 0
        lax.fori_loop(0, nb, zbody, 0)
        return 0
    lax.fori_loop(0, NCHUNK, chunk_body, 0)


@jax.jit
def kernel(pillar_features, voxel_coords):
    coords = jnp.asarray(voxel_coords, jnp.int32).reshape(-1)
    feat = jnp.asarray(pillar_features, jnp.float32)

    mesh = plsc.VectorSubcoreMesh(core_axis_name="c", subcore_axis_name="s")
    run = functools.partial(
        pl.kernel,
        out_type=jax.ShapeDtypeStruct((B * C * P,), jnp.float32),
        mesh=mesh,
        compiler_params=pltpu.CompilerParams(
            needs_layout_passes=False, use_tc_tiling_on_sc=False),
        scratch_types=[
            pltpu.VMEM((CH * 4,), jnp.int32),
            pltpu.VMEM((S,), jnp.int32),
            pltpu.VMEM((RMAX, C), jnp.float32),
            pltpu.VMEM((C * W,), jnp.float32),
            pltpu.VMEM((RMAX,), jnp.int32),
            pltpu.VMEM((RMAX,), jnp.int32),
            pltpu.SemaphoreType.DMA,
            pltpu.SemaphoreType.DMA,
        ],
    )(_scatter_kernel)
    out = run(feat, coords)
    return out.reshape(B, C * NZ, NY, NX)


# speedup vs baseline: 9.0483x; 1.0945x over previous
import functools

import jax
import jax.numpy as jnp
from jax import lax
from jax.experimental import pallas as pl
from jax.experimental.pallas import tpu as pltpu
from jax.experimental.pallas import tpu_sc as plsc

NX = 432
NY = 496
NZ = 1
C = 64
B = 4
N = 80000

NW = 32
YG = NY // 8
XT = 4
EDGE_W = NX - 3 * 128
NSTRIP = B * YG * XT
SPW = NSTRIP // NW
SCAP = 1024
INVW = SPW * SCAP
CH = 8000
CH4 = CH * 4
NCOORD = N // CH
RMAX = SCAP + 16
GB = 4


def _inv_kernel(coords_hbm, inv_hbm, coords_v, inv_v, csem):
    wid = lax.axis_index("s") * 2 + lax.axis_index("c")
    sbase = wid * SPW

    iota16 = lax.iota(jnp.int32, 16)
    zeros16i = jnp.zeros((16,), jnp.int32)

    def zero_body(k, _):
        inv_v[pl.ds(k * 16, 16)] = zeros16i
        return 0
    lax.fori_loop(0, INVW // 16, zero_body, 0)

    pltpu.async_copy(coords_hbm.at[pl.ds(0, CH4)],
                     coords_v.at[pl.ds(0, CH4)], csem)

    def coord_chunk(ci, _):
        off = (ci % 2) * CH4
        pltpu.make_async_copy(coords_hbm.at[pl.ds(ci * CH4, CH4)],
                              coords_v.at[pl.ds(off, CH4)], csem).wait()

        @pl.when(ci + 1 < NCOORD)
        def _():
            noff = ((ci + 1) % 2) * CH4
            pltpu.async_copy(coords_hbm.at[pl.ds((ci + 1) * CH4, CH4)],
                             coords_v.at[pl.ds(noff, CH4)], csem)

        def pillar_body(j, _):
            for u in range(4):
                rows = off + j * 256 + u * 64 + iota16 * 4
                bb = plsc.load_gather(coords_v, [rows])
                zz = plsc.load_gather(coords_v, [rows + 1])
                yy = plsc.load_gather(coords_v, [rows + 2])
                xx = plsc.load_gather(coords_v, [rows + 3])
                xx = xx + zz
                yg = yy >> 3
                y8 = yy & 7
                xt = xx >> 7
                xc = xx & 127
                strip = (bb * YG + yg) * 4 + xt
                local = (strip - sbase) * SCAP + (y8 << 7) + xc
                m = (strip >= sbase) & (strip < sbase + SPW)
                localc = jnp.minimum(jnp.maximum(local, 0), INVW - 1)
                ids = ci * CH + j * 64 + u * 16 + iota16 + 1
                plsc.store_scatter(inv_v, [localc], ids, mask=m)
            return 0
        lax.fori_loop(0, CH // 64, pillar_body, 0)
        return 0
    lax.fori_loop(0, NCOORD, coord_chunk, 0)

    pltpu.sync_copy(inv_v, inv_hbm.at[pl.ds(wid * INVW, INVW)])


def _write_kernel(feat_hbm, inv_hbm, out_hbm,
                  inv_s, plist, ilist, rows_v, tile_v, tile_e,
                  isem, gsem, wsem):
    wid = lax.axis_index("s") * 2 + lax.axis_index("c")

    iota16 = lax.iota(jnp.int32, 16)
    zeros16i = jnp.zeros((16,), jnp.int32)
    zeros16f = jnp.zeros((16,), jnp.float32)

    def tzero(r, _):
        for cc in range(8):
            tile_v[r, pl.ds(cc * 16, 16)] = zeros16f
        return 0
    lax.fori_loop(0, C * 8, tzero, 0)

    def tzero_e(r, _):
        for cc in range(3):
            tile_e[r, pl.ds(cc * 16, 16)] = zeros16f
        return 0
    lax.fori_loop(0, (C // 2) * 8, tzero_e, 0)

    pltpu.async_copy(inv_hbm.at[pl.ds(wid * SPW * SCAP, SCAP)],
                     inv_s.at[pl.ds(0, SCAP)], isem)

    def strip_body(s, _):
        g = wid * SPW + s
        xt = g & 3
        t = g >> 2
        yg = t % YG
        b = t // YG
        bc0 = b * C
        y0 = yg * 8
        ioff = (s % 2) * SCAP

        pltpu.make_async_copy(inv_hbm.at[pl.ds(g * SCAP, SCAP)],
                              inv_s.at[pl.ds(ioff, SCAP)], isem).wait()

        @pl.when(s + 1 < SPW)
        def _():
            noff = ((s + 1) % 2) * SCAP
            pltpu.async_copy(inv_hbm.at[pl.ds((g + 1) * SCAP, SCAP)],
                             inv_s.at[pl.ds(noff, SCAP)], isem)

        def cbody(k, cnt):
            inv16 = inv_s[pl.ds(ioff + k * 16, 16)]
            m = inv16 > 0
            plsc.store_compressed(plist.at[pl.ds(cnt, 16)],
                                  k * 16 + iota16, mask=m)
            plsc.store_compressed(ilist.at[pl.ds(cnt, 16)],
                                  inv16 - 1, mask=m)
            return cnt + jnp.sum(m.astype(jnp.int32))
        cnt = lax.fori_loop(0, SCAP // 16, cbody, 0)
        ilist[pl.ds(cnt, 16)] = zeros16i
        nb = (cnt + 15) // 16
        nsb = (nb + GB - 1) // GB

        def make_super_batch(tile_ref, gc_lo, gc_hi):
            def super_batch(t2, _):
                q0 = t2 * GB
                qe = jnp.minimum(q0 + GB, nb)

                def gfire(q, _):
                    idxv = ilist[pl.ds(q * 16, 16)]
                    pltpu.async_copy(
                        feat_hbm.at[idxv],
                        rows_v.at[pl.ds((q - q0) * 16, 16), :], gsem)
                    return 0
                lax.fori_loop(q0, qe, gfire, 0)

                def gdrain(q, _):
                    pltpu.make_async_copy(
                        feat_hbm.at[zeros16i],
                        rows_v.at[pl.ds((q - q0) * 16, 16), :], gsem).wait()
                    return 0
                lax.fori_loop(q0, qe, gdrain, 0)

                def sbody(q, _):
                    pos16 = plist[pl.ds(q * 16, 16)]
                    for lane in range(16):
                        ok16 = (zeros16i + q * 16 + lane) < cnt
                        pj = pos16[lane]
                        yj = pj >> 7
                        xj = pj & 127
                        row = (q - q0) * 16 + lane
                        for gc in range(gc_lo, gc_hi):
                            vals = rows_v[row, pl.ds(gc * 16, 16)]
                            rowv = ((gc - gc_lo) * 16 + iota16) * 8 + yj
                            plsc.store_scatter(tile_ref,
                                               [rowv, zeros16i + xj],
                                               vals, mask=ok16)
                    return 0
                lax.fori_loop(q0, qe, sbody, 0)
                return 0
            return super_batch

        def make_rezero(tile_ref, ngc):
            def zb(q, _):
                pos16 = plist[pl.ds(q * 16, 16)]
                for lane in range(16):
                    ok16 = (zeros16i + q * 16 + lane) < cnt
                    pj = pos16[lane]
                    yj = pj >> 7
                    xj = pj & 127
                    for gc in range(ngc):
                        rowv = (gc * 16 + iota16) * 8 + yj
                        plsc.store_scatter(tile_ref,
                                           [rowv, zeros16i + xj],
                                           zeros16f, mask=ok16)
                return 0
            return zb

        @pl.when(xt < 3)
        def _():
            lax.fori_loop(0, nsb, make_super_batch(tile_v, 0, 4), 0)
            x0 = xt * 128

            def wfire(c, _):
                pltpu.async_copy(tile_v.at[pl.ds(c * 8, 8), :],
                                 out_hbm.at[bc0 + c, pl.ds(y0, 8),
                                            pl.ds(x0, 128)], wsem)
                return 0
            lax.fori_loop(0, C, wfire, 0)
            pltpu.make_async_copy(feat_hbm.at[pl.ds(0, C * 8), :],
                                  tile_v, wsem).wait()
            lax.fori_loop(0, nb, make_rezero(tile_v, 4), 0)

        @pl.when(xt == 3)
        def _():
            for h in range(2):
                lax.fori_loop(0, nsb,
                              make_super_batch(tile_e, 2 * h, 2 * h + 2), 0)

                def wfire(c, _):
                    pltpu.async_copy(tile_e.at[pl.ds(c * 8, 8), :],
                                     out_hbm.at[bc0 + h * 32 + c,
                                                pl.ds(y0, 8),
                                                pl.ds(384, EDGE_W)], wsem)
                    return 0
                lax.fori_loop(0, C // 2, wfire, 0)

                def wdrain(c, _):
                    pltpu.make_async_copy(tile_e.at[pl.ds(c * 8, 8), :],
                                          out_hbm.at[bc0 + h * 32 + c,
                                                     pl.ds(y0, 8),
                                                     pl.ds(384, EDGE_W)],
                                          wsem).wait()
                    return 0
                lax.fori_loop(0, C // 2, wdrain, 0)
                lax.fori_loop(0, nb, make_rezero(tile_e, 2), 0)
        return 0
    lax.fori_loop(0, SPW, strip_body, 0)


@jax.jit
def kernel(pillar_features, voxel_coords):
    coords = jnp.asarray(voxel_coords, jnp.int32).reshape(-1)
    feat = jnp.asarray(pillar_features, jnp.float32)
    featp = jnp.pad(feat, ((0, 0), (0, 128 - C)))

    mesh = plsc.VectorSubcoreMesh(core_axis_name="c", subcore_axis_name="s")

    run1 = functools.partial(
        pl.kernel,
        out_type=jax.ShapeDtypeStruct((NSTRIP * SCAP,), jnp.int32),
        mesh=mesh,
        compiler_params=pltpu.CompilerParams(needs_layout_passes=False),
        scratch_types=[
            pltpu.VMEM((2 * CH4,), jnp.int32),
            pltpu.VMEM((INVW,), jnp.int32),
            pltpu.SemaphoreType.DMA,
        ],
    )(_inv_kernel)
    inv = run1(coords)

    run2 = functools.partial(
        pl.kernel,
        out_type=jax.ShapeDtypeStruct((B * C, NY, NX), jnp.float32),
        mesh=mesh,
        compiler_params=pltpu.CompilerParams(
            needs_layout_passes=False, use_tc_tiling_on_sc=True),
        scratch_types=[
            pltpu.VMEM((2 * SCAP,), jnp.int32),
            pltpu.VMEM((RMAX,), jnp.int32),
            pltpu.VMEM((RMAX,), jnp.int32),
            pltpu.VMEM((GB * 16, 128), jnp.float32),
            pltpu.VMEM((C * 8, 128), jnp.float32),
            pltpu.VMEM(((C // 2) * 8, EDGE_W), jnp.float32),
            pltpu.SemaphoreType.DMA,
            pltpu.SemaphoreType.DMA,
            pltpu.SemaphoreType.DMA,
        ],
    )(_write_kernel)
    out = run2(featp, inv)
    return out.reshape(B, C * NZ, NY, NX)


# speedup vs baseline: 20.4377x; 2.2587x over previous
import functools

import jax
import jax.numpy as jnp
from jax import lax
from jax.experimental import pallas as pl
from jax.experimental.pallas import tpu as pltpu
from jax.experimental.pallas import tpu_sc as plsc

NX = 432
NY = 496
NZ = 1
C = 64
B = 4
N = 80000

NW = 32
XG = NX // 8
NT = 4
EDGE_W = NY - 3 * 128
NSTRIP = B * XG * NT
SPW = NSTRIP // NW
SCAP = 1024
INVW = SPW * SCAP
CH = 8000
CH4 = CH * 4
NCOORD = N // CH
RMAX = SCAP + 16
GB = 4


def _inv_kernel(coords_hbm, inv_hbm, coords_v, inv_v, csem):
    wid = lax.axis_index("s") * 2 + lax.axis_index("c")
    sbase = wid * SPW

    iota16 = lax.iota(jnp.int32, 16)
    zeros16i = jnp.zeros((16,), jnp.int32)

    def zero_body(k, _):
        inv_v[pl.ds(k * 16, 16)] = zeros16i
        return 0
    lax.fori_loop(0, INVW // 16, zero_body, 0)

    pltpu.async_copy(coords_hbm.at[pl.ds(0, CH4)],
                     coords_v.at[pl.ds(0, CH4)], csem)

    def coord_chunk(ci, _):
        off = (ci % 2) * CH4
        pltpu.make_async_copy(coords_hbm.at[pl.ds(ci * CH4, CH4)],
                              coords_v.at[pl.ds(off, CH4)], csem).wait()

        @pl.when(ci + 1 < NCOORD)
        def _():
            noff = ((ci + 1) % 2) * CH4
            pltpu.async_copy(coords_hbm.at[pl.ds((ci + 1) * CH4, CH4)],
                             coords_v.at[pl.ds(noff, CH4)], csem)

        def pillar_body(j, _):
            for u in range(4):
                rows = off + j * 256 + u * 64 + iota16 * 4
                bb = plsc.load_gather(coords_v, [rows])
                zz = plsc.load_gather(coords_v, [rows + 1])
                yy = plsc.load_gather(coords_v, [rows + 2])
                xx = plsc.load_gather(coords_v, [rows + 3])
                xx = xx + zz
                xg = xx >> 3
                x8 = xx & 7
                yt = yy >> 7
                yc = yy & 127
                strip = (bb * XG + xg) * 4 + yt
                local = (strip - sbase) * SCAP + (x8 << 7) + yc
                m = (strip >= sbase) & (strip < sbase + SPW)
                localc = jnp.minimum(jnp.maximum(local, 0), INVW - 1)
                ids = ci * CH + j * 64 + u * 16 + iota16 + 1
                plsc.store_scatter(inv_v, [localc], ids, mask=m)
            return 0
        lax.fori_loop(0, CH // 64, pillar_body, 0)
        return 0
    lax.fori_loop(0, NCOORD, coord_chunk, 0)

    pltpu.sync_copy(inv_v, inv_hbm.at[pl.ds(wid * INVW, INVW)])


def _write_kernel(feat_hbm, inv_hbm, out_hbm,
                  inv_s, plist, ilist, rows_v, tile_v, tile_e,
                  isem, gsem, wsem):
    wid = lax.axis_index("s") * 2 + lax.axis_index("c")

    iota16 = lax.iota(jnp.int32, 16)
    zeros16i = jnp.zeros((16,), jnp.int32)
    zeros16f = jnp.zeros((16,), jnp.float32)

    def tzero(r, _):
        for cc in range(8):
            tile_v[r, pl.ds(cc * 16, 16)] = zeros16f
        return 0
    lax.fori_loop(0, C * 8, tzero, 0)

    def tzero_e(r, _):
        for cc in range(EDGE_W // 16):
            tile_e[r, pl.ds(cc * 16, 16)] = zeros16f
        return 0
    lax.fori_loop(0, (C // 2) * 8, tzero_e, 0)

    pltpu.async_copy(inv_hbm.at[pl.ds(wid * SPW * SCAP, SCAP)],
                     inv_s.at[pl.ds(0, SCAP)], isem)

    def strip_body(s, _):
        g = wid * SPW + s
        yt = g & 3
        t = g >> 2
        xg = t % XG
        b = t // XG
        bc0 = b * C
        x0 = xg * 8
        ioff = (s % 2) * SCAP

        pltpu.make_async_copy(inv_hbm.at[pl.ds(g * SCAP, SCAP)],
                              inv_s.at[pl.ds(ioff, SCAP)], isem).wait()

        @pl.when(s + 1 < SPW)
        def _():
            noff = ((s + 1) % 2) * SCAP
            pltpu.async_copy(inv_hbm.at[pl.ds((g + 1) * SCAP, SCAP)],
                             inv_s.at[pl.ds(noff, SCAP)], isem)

        def cbody(k, cnt):
            inv16 = inv_s[pl.ds(ioff + k * 16, 16)]
            m = inv16 > 0
            plsc.store_compressed(plist.at[pl.ds(cnt, 16)],
                                  k * 16 + iota16, mask=m)
            plsc.store_compressed(ilist.at[pl.ds(cnt, 16)],
                                  inv16 - 1, mask=m)
            return cnt + jnp.sum(m.astype(jnp.int32))
        cnt = lax.fori_loop(0, SCAP // 16, cbody, 0)
        ilist[pl.ds(cnt, 16)] = zeros16i
        nb = (cnt + 15) // 16
        nsb = (nb + GB - 1) // GB

        def make_super_batch(tile_ref, gc_lo, gc_hi):
            def super_batch(t2, _):
                q0 = t2 * GB
                qe = jnp.minimum(q0 + GB, nb)

                def gfire(q, _):
                    idxv = ilist[pl.ds(q * 16, 16)]
                    pltpu.async_copy(
                        feat_hbm.at[idxv],
                        rows_v.at[pl.ds((q - q0) * 16, 16), :], gsem)
                    return 0
                lax.fori_loop(q0, qe, gfire, 0)

                def gdrain(q, _):
                    pltpu.make_async_copy(
                        feat_hbm.at[zeros16i],
                        rows_v.at[pl.ds((q - q0) * 16, 16), :], gsem).wait()
                    return 0
                lax.fori_loop(q0, qe, gdrain, 0)

                def sbody(q, _):
                    pos16 = plist[pl.ds(q * 16, 16)]
                    for lane in range(16):
                        ok16 = (zeros16i + q * 16 + lane) < cnt
                        pj = pos16[lane]
                        yj = pj >> 7
                        xj = pj & 127
                        row = (q - q0) * 16 + lane
                        for gc in range(gc_lo, gc_hi):
                            vals = rows_v[row, pl.ds(gc * 16, 16)]
                            rowv = ((gc - gc_lo) * 16 + iota16) * 8 + yj
                            plsc.store_scatter(tile_ref,
                                               [rowv, zeros16i + xj],
                                               vals, mask=ok16)
                    return 0
                lax.fori_loop(q0, qe, sbody, 0)
                return 0
            return super_batch

        def make_rezero(tile_ref, ngc):
            def zb(q, _):
                pos16 = plist[pl.ds(q * 16, 16)]
                for lane in range(16):
                    ok16 = (zeros16i + q * 16 + lane) < cnt
                    pj = pos16[lane]
                    yj = pj >> 7
                    xj = pj & 127
                    for gc in range(ngc):
                        rowv = (gc * 16 + iota16) * 8 + yj
                        plsc.store_scatter(tile_ref,
                                           [rowv, zeros16i + xj],
                                           zeros16f, mask=ok16)
                return 0
            return zb

        @pl.when(yt < 3)
        def _():
            lax.fori_loop(0, nsb, make_super_batch(tile_v, 0, 4), 0)
            y0 = yt * 128

            def wfire(c, _):
                pltpu.async_copy(tile_v.at[pl.ds(c * 8, 8), :],
                                 out_hbm.at[bc0 + c, pl.ds(x0, 8),
                                            pl.ds(y0, 128)], wsem)
                return 0
            lax.fori_loop(0, C, wfire, 0)
            pltpu.make_async_copy(feat_hbm.at[pl.ds(0, C * 8), :],
                                  tile_v, wsem).wait()
            lax.fori_loop(0, nb, make_rezero(tile_v, 4), 0)

        @pl.when(yt == 3)
        def _():
            for h in range(2):
                lax.fori_loop(0, nsb,
                              make_super_batch(tile_e, 2 * h, 2 * h + 2), 0)

                def wfire(c, _):
                    pltpu.async_copy(tile_e.at[pl.ds(c * 8, 8), :],
                                     out_hbm.at[bc0 + h * 32 + c,
                                                pl.ds(x0, 8),
                                                pl.ds(384, EDGE_W)], wsem)
                    return 0
                lax.fori_loop(0, C // 2, wfire, 0)

                def wdrain(c, _):
                    pltpu.make_async_copy(tile_e.at[pl.ds(c * 8, 8), :],
                                          out_hbm.at[bc0 + h * 32 + c,
                                                     pl.ds(x0, 8),
                                                     pl.ds(384, EDGE_W)],
                                          wsem).wait()
                    return 0
                lax.fori_loop(0, C // 2, wdrain, 0)
                lax.fori_loop(0, nb, make_rezero(tile_e, 2), 0)
        return 0
    lax.fori_loop(0, SPW, strip_body, 0)


@jax.jit
def kernel(pillar_features, voxel_coords):
    coords = jnp.asarray(voxel_coords, jnp.int32).reshape(-1)
    feat = jnp.asarray(pillar_features, jnp.float32)
    featp = jnp.pad(feat, ((0, 0), (0, 128 - C)))

    mesh = plsc.VectorSubcoreMesh(core_axis_name="c", subcore_axis_name="s")

    run1 = functools.partial(
        pl.kernel,
        out_type=jax.ShapeDtypeStruct((NSTRIP * SCAP,), jnp.int32),
        mesh=mesh,
        compiler_params=pltpu.CompilerParams(needs_layout_passes=False),
        scratch_types=[
            pltpu.VMEM((2 * CH4,), jnp.int32),
            pltpu.VMEM((INVW,), jnp.int32),
            pltpu.SemaphoreType.DMA,
        ],
    )(_inv_kernel)
    inv = run1(coords)

    run2 = functools.partial(
        pl.kernel,
        out_type=jax.ShapeDtypeStruct((B * C, NX, NY), jnp.float32),
        mesh=mesh,
        compiler_params=pltpu.CompilerParams(
            needs_layout_passes=False, use_tc_tiling_on_sc=True),
        scratch_types=[
            pltpu.VMEM((2 * SCAP,), jnp.int32),
            pltpu.VMEM((RMAX,), jnp.int32),
            pltpu.VMEM((RMAX,), jnp.int32),
            pltpu.VMEM((GB * 16, 128), jnp.float32),
            pltpu.VMEM((C * 8, 128), jnp.float32),
            pltpu.VMEM(((C // 2) * 8, EDGE_W), jnp.float32),
            pltpu.SemaphoreType.DMA,
            pltpu.SemaphoreType.DMA,
            pltpu.SemaphoreType.DMA,
        ],
    )(_write_kernel)
    out = run2(featp, inv)
    return jnp.transpose(out.reshape(B, C * NZ, NX, NY), (0, 1, 3, 2))
